# baseline (speedup 1.0000x reference)
import jax
import jax.numpy as jnp
from jax import lax
from jax.experimental import pallas as pl
from jax.experimental.pallas import tpu as pltpu
from jax.experimental.pallas import tpu_sc as plsc

BATCH = 16384
IN_COLS = 100
OUT_COLS = 380
NUM_NONCAT = 60

NUM_WORKERS = 32
ROWS_PER_WORKER = BATCH // NUM_WORKERS
CHUNK_ROWS = 64
CHUNKS = ROWS_PER_WORKER // CHUNK_ROWS

_O_CARD4 = 61
_O_K8 = 77
_O_CARD8 = 80
_O_TAIL = 84


def _row_body(in_v, ob, rg, r, consts):
    (lane, b4, b8, b16, cls4, cls8, cls16, pat3, mask3, one, zero) = consts

    ld = {}
    for o in (0, 16, 32, 48, _O_CARD4, _O_K8, _O_CARD8, _O_TAIL):
        ld[o] = in_v[rg, pl.ds(o, 16)]

    def perm(v, idx):
        dn = lax.GatherDimensionNumbers(
            offset_dims=(), collapsed_slice_dims=(0,), start_index_map=(0,)
        )
        return lax.gather(
            v,
            idx[:, None],
            dimension_numbers=dn,
            slice_sizes=(1,),
            mode=lax.GatherScatterMode.PROMISE_IN_BOUNDS,
        )

    def onehot(src_o, pat, cls):
        s = perm(ld[src_o], pat)
        return jnp.where(s == cls, one, zero)

    ob[r, pl.ds(0, 16)] = ld[0]
    ob[r, pl.ds(16, 16)] = ld[16]
    ob[r, pl.ds(32, 16)] = ld[32]
    s3 = perm(ld[48], pat3)
    oh3 = jnp.where(s3 == cls4, one, zero)
    ob[r, pl.ds(48, 16)] = jnp.where(mask3, s3, oh3)
    for k in range(4, 8):
        pat = (60 + (16 * k - 60) // 4 - _O_CARD4) + b4
        ob[r, pl.ds(16 * k, 16)] = onehot(_O_CARD4, pat, cls4)
    ob[r, pl.ds(128, 16)] = onehot(_O_K8, b4, cls4)
    for k in range(9, 13):
        pat = (80 + (16 * k - 144) // 8 - _O_CARD8) + b8
        ob[r, pl.ds(16 * k, 16)] = onehot(_O_CARD8, pat, cls8)
    ob[r, pl.ds(208, 16)] = onehot(_O_TAIL, 4 + b8, cls8)
    for k in range(14, 23):
        pat = (90 + (16 * k - 224) // 16 - _O_TAIL) + b16
        ob[r, pl.ds(16 * k, 16)] = onehot(_O_TAIL, pat, cls16)
    pat23 = jnp.minimum((90 + (16 * 23 - 224) // 16 - _O_TAIL) + b16, 15)
    v23 = onehot(_O_TAIL, pat23, cls16)
    plsc.store_scatter(
        ob, [jnp.full((16,), r, jnp.int32), 368 + lane], v23, mask=lane < 12
    )


def _sc_kernel(
    x_hbm, out_hbm, in_v0, in_v1, out_v0, out_v1, semi0, semi1, semo0, semo1
):
    wid = lax.axis_index("s") * 2 + lax.axis_index("c")
    row0 = wid * ROWS_PER_WORKER

    pltpu.async_copy(x_hbm.at[pl.ds(row0, CHUNK_ROWS)], in_v0, semi0)
    pltpu.async_copy(
        x_hbm.at[pl.ds(row0 + CHUNK_ROWS, CHUNK_ROWS)], in_v1, semi1
    )

    lane = lax.iota(jnp.int32, 16)
    consts = (
        lane,
        lane >> 2,
        (lane + 4) >> 3,
        (lane + 4) >> 4,
        (lane & 3).astype(jnp.float32),
        ((lane + 4) & 7).astype(jnp.float32),
        ((lane + 4) & 15).astype(jnp.float32),
        jnp.minimum(lane, 12),
        lane < 12,
        jnp.full((16,), 1.0, jnp.float32),
        jnp.zeros((16,), jnp.float32),
    )

    bufs = ((in_v0, out_v0, semi0, semo0), (in_v1, out_v1, semi1, semo1))

    @pl.loop(0, CHUNKS, step=2)
    def _pair(c):
        for phase, (ib, ob, semi, semo) in enumerate(bufs):
            chunk = c + phase

            pltpu.make_async_copy(
                x_hbm.at[pl.ds(row0, CHUNK_ROWS)], ib, semi
            ).wait()

            @pl.when(chunk >= 2)
            def _wait_prev():
                pltpu.make_async_copy(
                    ob, out_hbm.at[pl.ds(row0, CHUNK_ROWS)], semo
                ).wait()

            @pl.loop(0, CHUNK_ROWS, step=2)
            def _row(r):
                _row_body(ib, ob, r, r, consts)
                _row_body(ib, ob, r + 1, r + 1, consts)

            pltpu.async_copy(
                ob,
                out_hbm.at[pl.ds(row0 + chunk * CHUNK_ROWS, CHUNK_ROWS)],
                semo,
            )

            @pl.when(chunk + 2 < CHUNKS)
            def _next_in():
                pltpu.async_copy(
                    x_hbm.at[pl.ds(row0 + (chunk + 2) * CHUNK_ROWS, CHUNK_ROWS)],
                    ib,
                    semi,
                )

    pltpu.make_async_copy(out_v0, out_hbm.at[pl.ds(row0, CHUNK_ROWS)], semo0).wait()
    pltpu.make_async_copy(out_v1, out_hbm.at[pl.ds(row0, CHUNK_ROWS)], semo1).wait()


@jax.jit
def kernel(x):
    mesh = plsc.VectorSubcoreMesh(core_axis_name="c", subcore_axis_name="s")
    f = pl.kernel(
        _sc_kernel,
        out_type=jax.ShapeDtypeStruct((BATCH, OUT_COLS), jnp.float32),
        mesh=mesh,
        scratch_types=[
            pltpu.VMEM((CHUNK_ROWS, IN_COLS), jnp.float32),
            pltpu.VMEM((CHUNK_ROWS, IN_COLS), jnp.float32),
            pltpu.VMEM((CHUNK_ROWS, OUT_COLS), jnp.float32),
            pltpu.VMEM((CHUNK_ROWS, OUT_COLS), jnp.float32),
            pltpu.SemaphoreType.DMA,
            pltpu.SemaphoreType.DMA,
            pltpu.SemaphoreType.DMA,
            pltpu.SemaphoreType.DMA,
        ],
        compiler_params=pltpu.CompilerParams(
            needs_layout_passes=False, use_tc_tiling_on_sc=True
        ),
    )
    return f(x)


# speedup vs baseline: 1.0063x; 1.0063x over previous
import jax
import jax.numpy as jnp
from jax import lax
from jax.experimental import pallas as pl
from jax.experimental.pallas import tpu as pltpu
from jax.experimental.pallas import tpu_sc as plsc

BATCH = 16384
IN_COLS = 100
OUT_COLS = 380
NUM_NONCAT = 60

NUM_WORKERS = 32
ROWS_PER_WORKER = BATCH // NUM_WORKERS
CHUNK_ROWS = 64
CHUNKS = ROWS_PER_WORKER // CHUNK_ROWS

_O_CARD4 = 61
_O_K8 = 77
_O_CARD8 = 80
_O_TAIL = 84


def _row_body(in_v, ob, rg, r, consts):
    (lane, b4, b8, b16, cls4, cls8, cls16, pat3, mask3, one, zero) = consts

    ld = {}
    for o in (0, 16, 32, 48, _O_CARD4, _O_K8, _O_CARD8, _O_TAIL):
        ld[o] = in_v[rg, pl.ds(o, 16)]

    def perm(v, idx):
        dn = lax.GatherDimensionNumbers(
            offset_dims=(), collapsed_slice_dims=(0,), start_index_map=(0,)
        )
        return lax.gather(
            v,
            idx[:, None],
            dimension_numbers=dn,
            slice_sizes=(1,),
            mode=lax.GatherScatterMode.PROMISE_IN_BOUNDS,
        )

    def onehot(src_o, pat, cls):
        s = perm(ld[src_o], pat)
        return jnp.where(s == cls, one, zero)

    ob[r, pl.ds(0, 16)] = ld[0]
    ob[r, pl.ds(16, 16)] = ld[16]
    ob[r, pl.ds(32, 16)] = ld[32]
    s3 = perm(ld[48], pat3)
    oh3 = jnp.where(s3 == cls4, one, zero)
    ob[r, pl.ds(48, 16)] = jnp.where(mask3, s3, oh3)
    for k in range(4, 8):
        pat = (60 + (16 * k - 60) // 4 - _O_CARD4) + b4
        ob[r, pl.ds(16 * k, 16)] = onehot(_O_CARD4, pat, cls4)
    ob[r, pl.ds(128, 16)] = onehot(_O_K8, b4, cls4)
    for k in range(9, 13):
        pat = (80 + (16 * k - 144) // 8 - _O_CARD8) + b8
        ob[r, pl.ds(16 * k, 16)] = onehot(_O_CARD8, pat, cls8)
    ob[r, pl.ds(208, 16)] = onehot(_O_TAIL, 4 + b8, cls8)
    for k in range(14, 23):
        pat = (90 + (16 * k - 224) // 16 - _O_TAIL) + b16
        ob[r, pl.ds(16 * k, 16)] = onehot(_O_TAIL, pat, cls16)
    pat23 = jnp.minimum((90 + (16 * 23 - 224) // 16 - _O_TAIL) + b16, 15)
    v23 = onehot(_O_TAIL, pat23, cls16)
    plsc.store_scatter(
        ob, [jnp.full((16,), r, jnp.int32), 368 + lane], v23, mask=lane < 12
    )


def _sc_kernel(
    x_hbm, out_hbm, in_v0, in_v1, out_v0, out_v1, semi0, semi1, semo0, semo1
):
    wid = lax.axis_index("s") * 2 + lax.axis_index("c")
    row0 = wid * ROWS_PER_WORKER

    pltpu.async_copy(x_hbm.at[pl.ds(row0, CHUNK_ROWS)], in_v0, semi0)
    pltpu.async_copy(
        x_hbm.at[pl.ds(row0 + CHUNK_ROWS, CHUNK_ROWS)], in_v1, semi1
    )

    lane = lax.iota(jnp.int32, 16)
    consts = (
        lane,
        lane >> 2,
        (lane + 4) >> 3,
        (lane + 4) >> 4,
        (lane & 3).astype(jnp.float32),
        ((lane + 4) & 7).astype(jnp.float32),
        ((lane + 4) & 15).astype(jnp.float32),
        jnp.minimum(lane, 12),
        lane < 12,
        jnp.full((16,), 1.0, jnp.float32),
        jnp.zeros((16,), jnp.float32),
    )

    bufs = ((in_v0, out_v0, semi0, semo0), (in_v1, out_v1, semi1, semo1))

    @pl.loop(0, CHUNKS, step=2)
    def _pair(c):
        for phase, (ib, ob, semi, semo) in enumerate(bufs):
            chunk = c + phase

            pltpu.make_async_copy(
                x_hbm.at[pl.ds(row0, CHUNK_ROWS)], ib, semi
            ).wait()

            @pl.when(chunk >= 2)
            def _wait_prev():
                pltpu.make_async_copy(
                    ob, out_hbm.at[pl.ds(row0, CHUNK_ROWS)], semo
                ).wait()

            @pl.loop(0, CHUNK_ROWS)
            def _row(r):
                _row_body(ib, ob, r, r, consts)

            pltpu.async_copy(
                ob,
                out_hbm.at[pl.ds(row0 + chunk * CHUNK_ROWS, CHUNK_ROWS)],
                semo,
            )

            @pl.when(chunk + 2 < CHUNKS)
            def _next_in():
                pltpu.async_copy(
                    x_hbm.at[pl.ds(row0 + (chunk + 2) * CHUNK_ROWS, CHUNK_ROWS)],
                    ib,
                    semi,
                )

    pltpu.make_async_copy(out_v0, out_hbm.at[pl.ds(row0, CHUNK_ROWS)], semo0).wait()
    pltpu.make_async_copy(out_v1, out_hbm.at[pl.ds(row0, CHUNK_ROWS)], semo1).wait()


@jax.jit
def kernel(x):
    mesh = plsc.VectorSubcoreMesh(core_axis_name="c", subcore_axis_name="s")
    f = pl.kernel(
        _sc_kernel,
        out_type=jax.ShapeDtypeStruct((BATCH, OUT_COLS), jnp.float32),
        mesh=mesh,
        scratch_types=[
            pltpu.VMEM((CHUNK_ROWS, IN_COLS), jnp.float32),
            pltpu.VMEM((CHUNK_ROWS, IN_COLS), jnp.float32),
            pltpu.VMEM((CHUNK_ROWS, OUT_COLS), jnp.float32),
            pltpu.VMEM((CHUNK_ROWS, OUT_COLS), jnp.float32),
            pltpu.SemaphoreType.DMA,
            pltpu.SemaphoreType.DMA,
            pltpu.SemaphoreType.DMA,
            pltpu.SemaphoreType.DMA,
        ],
        compiler_params=pltpu.CompilerParams(
            needs_layout_passes=False, use_tc_tiling_on_sc=True
        ),
    )
    return f(x)


# speedup vs baseline: 1.0101x; 1.0037x over previous
import jax
import jax.numpy as jnp
from jax import lax
from jax.experimental import pallas as pl
from jax.experimental.pallas import tpu as pltpu
from jax.experimental.pallas import tpu_sc as plsc

BATCH = 16384
IN_COLS = 100
OUT_COLS = 380
NUM_NONCAT = 60

NUM_WORKERS = 32
ROWS_PER_WORKER = BATCH // NUM_WORKERS
CHUNK_ROWS = 64
CHUNKS = ROWS_PER_WORKER // CHUNK_ROWS

_O_CARD4 = 61
_O_K8 = 77
_O_CARD8 = 80
_O_TAIL = 84


def _row_body(in_v, ob, rg, r, consts):
    (lane, b4, b8, b16, cls4, cls8, cls16, pat3, mask3, one, zero) = consts

    ld = {}
    for o in (0, 16, 32, 48, _O_CARD4, _O_K8, _O_CARD8, _O_TAIL):
        ld[o] = in_v[rg, pl.ds(o, 16)]

    def perm(v, idx):
        dn = lax.GatherDimensionNumbers(
            offset_dims=(), collapsed_slice_dims=(0,), start_index_map=(0,)
        )
        return lax.gather(
            v,
            idx[:, None],
            dimension_numbers=dn,
            slice_sizes=(1,),
            mode=lax.GatherScatterMode.PROMISE_IN_BOUNDS,
        )

    def onehot(src_o, pat, cls):
        s = perm(ld[src_o], pat)
        return jnp.where(s == cls, one, zero)

    ob[r, pl.ds(0, 16)] = ld[0]
    ob[r, pl.ds(16, 16)] = ld[16]
    ob[r, pl.ds(32, 16)] = ld[32]
    s3 = perm(ld[48], pat3)
    oh3 = jnp.where(s3 == cls4, one, zero)
    ob[r, pl.ds(48, 16)] = jnp.where(mask3, s3, oh3)
    for k in range(4, 8):
        pat = (60 + (16 * k - 60) // 4 - _O_CARD4) + b4
        ob[r, pl.ds(16 * k, 16)] = onehot(_O_CARD4, pat, cls4)
    ob[r, pl.ds(128, 16)] = onehot(_O_K8, b4, cls4)
    for k in range(9, 13):
        pat = (80 + (16 * k - 144) // 8 - _O_CARD8) + b8
        ob[r, pl.ds(16 * k, 16)] = onehot(_O_CARD8, pat, cls8)
    ob[r, pl.ds(208, 16)] = onehot(_O_TAIL, 4 + b8, cls8)
    for k in range(14, 23):
        pat = (90 + (16 * k - 224) // 16 - _O_TAIL) + b16
        ob[r, pl.ds(16 * k, 16)] = onehot(_O_TAIL, pat, cls16)
    pat23 = jnp.minimum((90 + (16 * 23 - 224) // 16 - _O_TAIL) + b16, 15)
    v23 = onehot(_O_TAIL, pat23, cls16)
    plsc.store_scatter(
        ob, [jnp.full((16,), r, jnp.int32), 368 + lane], v23, mask=lane < 12
    )


def _sc_kernel(
    x_hbm, out_hbm, in_v0, in_v1, out_v0, out_v1, semi0, semi1, semo0, semo1
):
    wid = lax.axis_index("s") * 2 + lax.axis_index("c")
    row0 = wid * ROWS_PER_WORKER

    pltpu.async_copy(x_hbm.at[pl.ds(row0, CHUNK_ROWS)], in_v0, semi0)
    pltpu.async_copy(
        x_hbm.at[pl.ds(row0 + CHUNK_ROWS, CHUNK_ROWS)], in_v1, semi1
    )

    lane = lax.iota(jnp.int32, 16)
    consts = (
        lane,
        lane >> 2,
        (lane + 4) >> 3,
        (lane + 4) >> 4,
        (lane & 3).astype(jnp.float32),
        ((lane + 4) & 7).astype(jnp.float32),
        ((lane + 4) & 15).astype(jnp.float32),
        jnp.minimum(lane, 12),
        lane < 12,
        jnp.full((16,), 1.0, jnp.float32),
        jnp.zeros((16,), jnp.float32),
    )

    bufs = ((in_v0, out_v0, semi0, semo0), (in_v1, out_v1, semi1, semo1))

    @pl.loop(0, CHUNKS, step=2)
    def _pair(c):
        for phase, (ib, ob, semi, semo) in enumerate(bufs):
            chunk = c + phase

            pltpu.make_async_copy(
                x_hbm.at[pl.ds(row0, CHUNK_ROWS)], ib, semi
            ).wait()

            @pl.when(chunk >= 2)
            def _wait_prev():
                pltpu.make_async_copy(
                    ob, out_hbm.at[pl.ds(row0, CHUNK_ROWS)], semo
                ).wait()

            @pl.loop(0, CHUNK_ROWS)
            def _row(r):
                _row_body(ib, ob, r, r, consts)

            pltpu.async_copy(
                ob,
                out_hbm.at[pl.ds(row0 + chunk * CHUNK_ROWS, CHUNK_ROWS)],
                semo,
            )

            @pl.when(chunk + 2 < CHUNKS)
            def _next_in():
                pltpu.async_copy(
                    x_hbm.at[pl.ds(row0 + (chunk + 2) * CHUNK_ROWS, CHUNK_ROWS)],
                    ib,
                    semi,
                )

    pltpu.make_async_copy(out_v0, out_hbm.at[pl.ds(row0, CHUNK_ROWS)], semo0).wait()
    pltpu.make_async_copy(out_v1, out_hbm.at[pl.ds(row0, CHUNK_ROWS)], semo1).wait()


@jax.jit
def kernel(x):
    mesh = plsc.VectorSubcoreMesh(core_axis_name="c", subcore_axis_name="s")
    f = pl.kernel(
        _sc_kernel,
        out_type=jax.ShapeDtypeStruct((BATCH, OUT_COLS), jnp.float32),
        mesh=mesh,
        scratch_types=[
            pltpu.VMEM((CHUNK_ROWS, IN_COLS), jnp.float32),
            pltpu.VMEM((CHUNK_ROWS, IN_COLS), jnp.float32),
            pltpu.VMEM((CHUNK_ROWS, OUT_COLS), jnp.float32),
            pltpu.VMEM((CHUNK_ROWS, OUT_COLS), jnp.float32),
            pltpu.SemaphoreType.DMA,
            pltpu.SemaphoreType.DMA,
            pltpu.SemaphoreType.DMA,
            pltpu.SemaphoreType.DMA,
        ],
        compiler_params=pltpu.CompilerParams(
            needs_layout_passes=False,
            use_tc_tiling_on_sc=True,
            disable_bounds_checks=True,
            skip_device_barrier=True,
        ),
    )
    return f(x)
